# Initial kernel scaffold; baseline (speedup 1.0000x reference)
import jax
import jax.numpy as jnp
from jax.experimental import pallas as pl


def kernel(x, emb_table, W1, b1, W2, b2, Wo, bo):
    raise NotImplementedError("write your pallas kernel here")


# speedup vs baseline: 1.0349x; 1.0349x over previous
import functools

import jax
import jax.numpy as jnp
from jax import lax
from jax.experimental import pallas as pl
from jax.experimental.pallas import tpu as pltpu
from jax.experimental.pallas import tpu_sc as plsc

_EMB = 64
_HID = 128
_TAGS = 5
_BATCH = 4096
_SEQ = 200
_NW = 32
_RPW = _BATCH // _NW
_C0 = 128
_C1 = _SEQ - _C0


def _pool_sc(x, emb_table):
  mesh = plsc.VectorSubcoreMesh(core_axis_name="c", subcore_axis_name="s",
                                num_cores=2, num_subcores=16)

  @functools.partial(
      pl.kernel,
      out_type=jax.ShapeDtypeStruct((_BATCH, _EMB), jnp.float32),
      mesh=mesh,
      compiler_params=pltpu.CompilerParams(use_tc_tiling_on_sc=False),
      scratch_types=[
          pltpu.VMEM((_RPW, _SEQ), jnp.int32),
          pltpu.VMEM((_SEQ, _EMB), jnp.float32),
          pltpu.VMEM((_SEQ, _EMB), jnp.float32),
          pltpu.VMEM((_RPW, _EMB), jnp.float32),
          pltpu.SemaphoreType.DMA,
          pltpu.SemaphoreType.DMA,
      ],
  )
  def k(x_hbm, tab_hbm, out_hbm, idx_v, rows0, rows1, out_v, sem0, sem1):
    wid = lax.axis_index("s") * 2 + lax.axis_index("c")
    base = wid * _RPW
    pltpu.sync_copy(x_hbm.at[pl.ds(base, _RPW), :], idx_v)

    def start(r, buf, sem):
      pltpu.async_copy(tab_hbm.at[idx_v.at[r, pl.ds(0, _C0)]],
                       buf.at[pl.ds(0, _C0), :], sem)
      pltpu.async_copy(tab_hbm.at[idx_v.at[r, pl.ds(_C0, _C1)]],
                       buf.at[pl.ds(_C0, _C1), :], sem)

    def wait(buf, sem):
      pltpu.make_async_copy(tab_hbm.at[pl.ds(0, _SEQ), :], buf, sem).wait()

    def compute(r, buf):
      def sbody(j, accs):
        a0, a1, a2, a3 = accs
        a0 = a0 + buf[j, pl.ds(0, 16)]
        a1 = a1 + buf[j, pl.ds(16, 16)]
        a2 = a2 + buf[j, pl.ds(32, 16)]
        a3 = a3 + buf[j, pl.ds(48, 16)]
        return (a0, a1, a2, a3)
      z = jnp.zeros((16,), jnp.float32)
      a0, a1, a2, a3 = lax.fori_loop(0, _SEQ, sbody, (z, z, z, z))
      out_v[r, pl.ds(0, 16)] = a0
      out_v[r, pl.ds(16, 16)] = a1
      out_v[r, pl.ds(32, 16)] = a2
      out_v[r, pl.ds(48, 16)] = a3

    start(0, rows0, sem0)

    def body(i, carry):
      r = 2 * i
      start(r + 1, rows1, sem1)
      wait(rows0, sem0)
      compute(r, rows0)
      start(r + 2, rows0, sem0)
      wait(rows1, sem1)
      compute(r + 1, rows1)
      return carry

    lax.fori_loop(0, (_RPW - 2) // 2, body, 0)
    start(_RPW - 1, rows1, sem1)
    wait(rows0, sem0)
    compute(_RPW - 2, rows0)
    wait(rows1, sem1)
    compute(_RPW - 1, rows1)

    pltpu.sync_copy(out_v, out_hbm.at[pl.ds(base, _RPW), :])

  return k(x, emb_table)


def _mlp_body(x_ref, s_ref, w1_ref, b1_ref, w2_ref, b2_ref, wo_ref, bo_ref,
              o_ref):
  cnt = jnp.sum((x_ref[...] != 0).astype(jnp.float32), axis=1, keepdims=True)
  h = s_ref[...] / (cnt + 1e-10)
  z = jnp.dot(h, w1_ref[...], preferred_element_type=jnp.float32)
  z = jnp.maximum(z + b1_ref[...], 0.0)
  z = jnp.dot(z, w2_ref[...], preferred_element_type=jnp.float32)
  z = jnp.maximum(z + b2_ref[...], 0.0)
  o_ref[...] = jnp.dot(z, wo_ref[...], preferred_element_type=jnp.float32) + bo_ref[...]


def _mlp_tc(x, s, W1, b1, W2, b2, Wo, bo):
  return pl.pallas_call(
      _mlp_body,
      out_shape=jax.ShapeDtypeStruct((_BATCH, _TAGS), jnp.float32),
  )(x, s, W1, b1.reshape(1, _HID), W2, b2.reshape(1, _HID), Wo,
    bo.reshape(1, _TAGS))


@jax.jit
def kernel(x, emb_table, W1, b1, W2, b2, Wo, bo):
  s = _pool_sc(x, emb_table)
  return _mlp_tc(x, s, W1, b1, W2, b2, Wo, bo)
